# Initial kernel scaffold; baseline (speedup 1.0000x reference)
import jax
import jax.numpy as jnp
from jax.experimental import pallas as pl


def kernel(trait_values, trait_confidences, trait_indices, emb_table, Wv, bv, Wc, bc, gamma, beta):
    raise NotImplementedError("write your pallas kernel here")


# speedup vs baseline: 2.0047x; 2.0047x over previous
import functools

import jax
import jax.numpy as jnp
from jax import lax
from jax.experimental import pallas as pl
from jax.experimental.pallas import tpu as pltpu
from jax.experimental.pallas import tpu_sc as plsc

_CH = 128


@functools.lru_cache(maxsize=None)
def _make_sc_gather(V: int, D2: int, N: int, NC: int, NS: int):
    NW = NC * NS
    n_per_w = N // NW
    n_ch = n_per_w // _CH
    mesh = plsc.VectorSubcoreMesh(core_axis_name="c", subcore_axis_name="s")

    @functools.partial(
        pl.kernel,
        mesh=mesh,
        out_type=jax.ShapeDtypeStruct((N, D2), jnp.float32),
        scratch_types=[
            pltpu.VMEM((n_ch, _CH), jnp.int32),
            pltpu.VMEM((_CH, D2), jnp.float32),
            pltpu.SemaphoreType.DMA,
        ],
    )
    def gather_k(table_hbm, idx_hbm, out_hbm, idx_v, buf, sem):
        wid = lax.axis_index("s") * NC + lax.axis_index("c")
        base = wid * n_per_w
        pltpu.sync_copy(idx_hbm.at[wid], idx_v)

        def body(j, carry):
            pltpu.async_copy(table_hbm.at[idx_v.at[j]], buf, sem).wait()
            pltpu.sync_copy(buf, out_hbm.at[pl.ds(base + j * _CH, _CH)])
            return carry

        lax.fori_loop(0, n_ch, body, 0)

    return gather_k


def _tc_body(g_ref, vc_ref, Wc_ref, Wv_ref, bv_ref, bc_ref, gm_ref, bt_ref,
             o_ref):
    D = o_ref.shape[-1]
    dn = (((1,), (1,)), ((), ()))
    g = g_ref[..., :D]
    ve = lax.dot_general(vc_ref[...], Wv_ref[...], dn,
                         preferred_element_type=jnp.float32) + bv_ref[...]
    y = (lax.dot_general(g, Wc_ref[..., :D], dn,
                         preferred_element_type=jnp.float32)
         + lax.dot_general(ve, Wc_ref[..., D:], dn,
                           preferred_element_type=jnp.float32)
         + bc_ref[...])
    mu = jnp.mean(y, axis=-1, keepdims=True)
    yc = y - mu
    var = jnp.mean(yc * yc, axis=-1, keepdims=True)
    o_ref[...] = yc * lax.rsqrt(var + 1e-5) * gm_ref[...] + bt_ref[...]


def _dense_tail(gathered, vc, Wc, Wv, bv, bc, gamma, beta, rows_blk):
    N, D2 = gathered.shape
    D = D2 // 2
    grid = (N // rows_blk,)
    small = lambda shp: pl.BlockSpec(shp, lambda i: (0, 0))
    return pl.pallas_call(
        _tc_body,
        grid=grid,
        in_specs=[
            pl.BlockSpec((rows_blk, D2), lambda i: (i, 0)),
            pl.BlockSpec((rows_blk, 2), lambda i: (i, 0)),
            small(Wc.shape),
            small(Wv.shape),
            small((1, D)),
            small((1, D)),
            small((1, D)),
            small((1, D)),
        ],
        out_specs=pl.BlockSpec((rows_blk, D), lambda i: (i, 0)),
        out_shape=jax.ShapeDtypeStruct((N, D), jnp.float32),
    )(gathered, vc, Wc, Wv, bv.reshape(1, D), bc.reshape(1, D),
      gamma.reshape(1, D), beta.reshape(1, D))


def kernel(trait_values, trait_confidences, trait_indices, emb_table,
           Wv, bv, Wc, bc, gamma, beta):
    B, T = trait_values.shape
    V, D = emb_table.shape
    N = B * T
    info = plsc.get_sparse_core_info()
    NW = info.num_cores * info.num_subcores
    idx3d = trait_indices.reshape(NW, N // (NW * _CH), _CH).astype(jnp.int32)
    table_pad = jnp.concatenate(
        [emb_table, jnp.zeros((V, D), jnp.float32)], axis=1)
    gathered = _make_sc_gather(V, 2 * D, N, info.num_cores,
                               info.num_subcores)(table_pad, idx3d)
    vc = jnp.stack(
        [trait_values.reshape(N), trait_confidences.reshape(N)], axis=-1)
    out = _dense_tail(gathered, vc, Wc, Wv, bv, bc, gamma, beta, 2048)
    return out.reshape(B, T, D)
